# baseline (speedup 1.0000x reference)
import functools

import jax
import jax.numpy as jnp
from jax import lax
from jax.experimental import pallas as pl
from jax.experimental.pallas import tpu as pltpu
from jax.experimental.pallas import tpu_sc as plsc

K_CLIP = 4
SIZE = 64
BATCH = 4096
HIST = 200
N = BATCH * HIST
NP = N // 2

NUM_CORES = 2
NUM_SUBCORES = 16
NW = NUM_CORES * NUM_SUBCORES
P_PER_W = NP // NW
CHUNK = 512
N_CHUNKS = P_PER_W // CHUNK
T2_ROWS = 88

_mesh = plsc.VectorSubcoreMesh(core_axis_name="c", subcore_axis_name="s")


def _vgather(vals, idx):
    dnums = lax.GatherDimensionNumbers(
        offset_dims=(), collapsed_slice_dims=(0,), start_index_map=(0,))
    return lax.gather(vals, idx[:, None], dnums, (1,),
                      mode=lax.GatherScatterMode.PROMISE_IN_BOUNDS)


@functools.partial(
    pl.kernel,
    mesh=_mesh,
    out_type=(
        jax.ShapeDtypeStruct((NP, 2 * SIZE), jnp.float32),
        jax.ShapeDtypeStruct((NUM_CORES * T2_ROWS, 2 * SIZE), jnp.float32),
    ),
    scratch_types=[
        pltpu.VMEM((9, SIZE), jnp.float32),
        pltpu.VMEM((T2_ROWS, 2 * SIZE), jnp.float32),
        pltpu.VMEM((2 * CHUNK,), jnp.int32),
        pltpu.VMEM((CHUNK,), jnp.int32),
        pltpu.VMEM((CHUNK, 2 * SIZE), jnp.float32),
        pltpu.SemaphoreType.DMA,
    ],
)
def _sc_lookup(idx_hbm, table_hbm, out_hbm, t2_hbm,
               tv, t2v, idx_v, pidx_v, rows_v, sem):
    c = lax.axis_index("c")
    s = lax.axis_index("s")
    wid = s * NUM_CORES + c
    t2_base = c * T2_ROWS

    @pl.when(s == 0)
    def _build():
        pltpu.sync_copy(table_hbm, tv)

        def row_body(i, carry):
            a = i // 9
            b = i - a * 9

            def q_body(q, carry2):
                t2v[i, pl.ds(q * 16, 16)] = tv[a, pl.ds(q * 16, 16)]
                t2v[i, pl.ds(SIZE + q * 16, 16)] = tv[b, pl.ds(q * 16, 16)]
                return carry2

            lax.fori_loop(0, SIZE // 16, q_body, 0)
            return carry

        lax.fori_loop(0, 81, row_body, 0)
        pltpu.sync_copy(t2v, t2_hbm.at[pl.ds(t2_base, T2_ROWS)])

    plsc.subcore_barrier()

    base0 = wid * P_PER_W
    lane = lax.iota(jnp.int32, 16)

    def chunk_body(g, carry):
        base = base0 + g * CHUNK
        pltpu.sync_copy(idx_hbm.at[pl.ds(2 * base, 2 * CHUNK)], idx_v)

        def fuse_body(k, carry2):
            w0 = idx_v[pl.ds(32 * k, 16)]
            w1 = idx_v[pl.ds(32 * k + 16, 16)]
            lo8 = lane < 8
            p_e0 = jnp.minimum(2 * lane, 14)
            p_e1 = jnp.maximum(2 * lane - 16, 0)
            p_o0 = jnp.minimum(2 * lane + 1, 15)
            p_o1 = jnp.maximum(2 * lane - 15, 1)
            ev = jnp.where(lo8, _vgather(w0, p_e0), _vgather(w1, p_e1))
            od = jnp.where(lo8, _vgather(w0, p_o0), _vgather(w1, p_o1))
            ev = jnp.minimum(jnp.maximum(ev, -K_CLIP), K_CLIP) + K_CLIP
            od = jnp.minimum(jnp.maximum(od, -K_CLIP), K_CLIP) + K_CLIP
            pidx_v[pl.ds(k * 16, 16)] = t2_base + ev * 9 + od
            return carry2

        lax.fori_loop(0, CHUNK // 16, fuse_body, 0)
        pltpu.async_copy(t2_hbm.at[pidx_v], rows_v, sem).wait()
        pltpu.sync_copy(rows_v, out_hbm.at[pl.ds(base, CHUNK)])
        return carry

    lax.fori_loop(0, N_CHUNKS, chunk_body, 0)


def kernel(input, table):
    out, _ = _sc_lookup(input.reshape(-1), table)
    return out.reshape(BATCH, HIST, SIZE)


# speedup vs baseline: 6.5605x; 6.5605x over previous
import functools

import jax
import jax.numpy as jnp
from jax import lax
from jax.experimental import pallas as pl
from jax.experimental.pallas import tpu as pltpu
from jax.experimental.pallas import tpu_sc as plsc

K_CLIP = 4
SIZE = 64
BATCH = 4096
HIST = 200
N = BATCH * HIST
NP = N // 2

NUM_CORES = 2
NUM_SUBCORES = 16
NW = NUM_CORES * NUM_SUBCORES
P_PER_W = NP // NW
CHUNK = 400
N_CHUNKS = P_PER_W // CHUNK
T2_ROWS = 88

_mesh = plsc.VectorSubcoreMesh(core_axis_name="c", subcore_axis_name="s")


def _vgather(vals, idx):
    dnums = lax.GatherDimensionNumbers(
        offset_dims=(), collapsed_slice_dims=(0,), start_index_map=(0,))
    return lax.gather(vals, idx[:, None], dnums, (1,),
                      mode=lax.GatherScatterMode.PROMISE_IN_BOUNDS)


@functools.partial(
    pl.kernel,
    mesh=_mesh,
    out_type=jax.ShapeDtypeStruct((NP, 2 * SIZE), jnp.float32),
    scratch_types=[
        pltpu.VMEM_SHARED((T2_ROWS, 2 * SIZE), jnp.float32),
        pltpu.VMEM((9, SIZE), jnp.float32),
        pltpu.VMEM((T2_ROWS, 2 * SIZE), jnp.float32),
        pltpu.VMEM((2 * CHUNK,), jnp.int32),
        pltpu.VMEM((2 * CHUNK,), jnp.int32),
        pltpu.VMEM((CHUNK,), jnp.int32),
        pltpu.VMEM((CHUNK,), jnp.int32),
        pltpu.VMEM((CHUNK, 2 * SIZE), jnp.float32),
        pltpu.VMEM((CHUNK, 2 * SIZE), jnp.float32),
        pltpu.SemaphoreType.DMA,
        pltpu.SemaphoreType.DMA,
    ],
)
def _sc_lookup(idx_hbm, table_hbm, out_hbm,
               t2_sh, tv, t2v, idx0, idx1, pidx0, pidx1,
               rows0, rows1, gsem, wsem):
    c = lax.axis_index("c")
    s = lax.axis_index("s")
    wid = s * NUM_CORES + c

    @pl.when(s == 0)
    def _build():
        pltpu.sync_copy(table_hbm, tv)

        def row_body(i, carry):
            a = i // 9
            b = i - a * 9

            def q_body(q, carry2):
                t2v[i, pl.ds(q * 16, 16)] = tv[a, pl.ds(q * 16, 16)]
                t2v[i, pl.ds(SIZE + q * 16, 16)] = tv[b, pl.ds(q * 16, 16)]
                return carry2

            lax.fori_loop(0, SIZE // 16, q_body, 0)
            return carry

        lax.fori_loop(0, 81, row_body, 0)
        pltpu.sync_copy(t2v, t2_sh)

    plsc.subcore_barrier()

    base0 = wid * P_PER_W
    lane = lax.iota(jnp.int32, 16)
    rows = (rows0, rows1)
    idxs = (idx0, idx1)
    pidxs = (pidx0, pidx1)

    def prep(g):
        idx_v = idxs[g % 2]
        pidx_v = pidxs[g % 2]
        base = base0 + g * CHUNK
        pltpu.sync_copy(idx_hbm.at[pl.ds(2 * base, 2 * CHUNK)], idx_v)

        def fuse_body(k, carry):
            w0 = idx_v[pl.ds(32 * k, 16)]
            w1 = idx_v[pl.ds(32 * k + 16, 16)]
            lo8 = lane < 8
            ev = jnp.where(lo8,
                           _vgather(w0, jnp.minimum(2 * lane, 14)),
                           _vgather(w1, jnp.maximum(2 * lane - 16, 0)))
            od = jnp.where(lo8,
                           _vgather(w0, jnp.minimum(2 * lane + 1, 15)),
                           _vgather(w1, jnp.maximum(2 * lane - 15, 1)))
            ev = jnp.minimum(jnp.maximum(ev, -K_CLIP), K_CLIP) + K_CLIP
            od = jnp.minimum(jnp.maximum(od, -K_CLIP), K_CLIP) + K_CLIP
            pidx_v[pl.ds(k * 16, 16)] = ev * 9 + od
            return carry

        lax.fori_loop(0, CHUNK // 16, fuse_body, 0)

    def gather_start(g):
        pltpu.async_copy(t2_sh.at[pidxs[g % 2]], rows[g % 2], gsem)

    def gather_wait(g):
        pltpu.make_async_copy(t2_sh.at[pidxs[g % 2]], rows[g % 2],
                              gsem).wait()

    def write_start(g):
        base = base0 + g * CHUNK
        pltpu.async_copy(rows[g % 2], out_hbm.at[pl.ds(base, CHUNK)], wsem)

    def write_wait(g):
        base = base0 + g * CHUNK
        pltpu.make_async_copy(rows[g % 2], out_hbm.at[pl.ds(base, CHUNK)],
                              wsem).wait()

    prep(0)
    gather_start(0)
    prep(1)
    gather_start(1)
    for g in range(N_CHUNKS):
        gather_wait(g)
        write_start(g)
        if g + 2 < N_CHUNKS:
            prep(g + 2)
            write_wait(g)
            gather_start(g + 2)
    write_wait(N_CHUNKS - 2)
    write_wait(N_CHUNKS - 1)


def kernel(input, table):
    out = _sc_lookup(input.reshape(-1), table)
    return out.reshape(BATCH, HIST, SIZE)


# speedup vs baseline: 9.4034x; 1.4333x over previous
import functools

import jax
import jax.numpy as jnp
from jax import lax
from jax.experimental import pallas as pl
from jax.experimental.pallas import tpu as pltpu
from jax.experimental.pallas import tpu_sc as plsc

K_CLIP = 4
SIZE = 64
BATCH = 4096
HIST = 200
N = BATCH * HIST

NUM_CORES = 2
NUM_SUBCORES = 16
NW = NUM_CORES * NUM_SUBCORES
B_PER_W = N // NW
CHUNK = 400
N_CHUNKS = B_PER_W // CHUNK
T1_ROWS = 16

_mesh = plsc.VectorSubcoreMesh(core_axis_name="c", subcore_axis_name="s")


@functools.partial(
    pl.kernel,
    mesh=_mesh,
    out_type=jax.ShapeDtypeStruct((N, SIZE), jnp.float32),
    scratch_types=[
        pltpu.VMEM_SHARED((T1_ROWS, SIZE), jnp.float32),
        pltpu.VMEM((T1_ROWS, SIZE), jnp.float32),
        pltpu.VMEM((CHUNK,), jnp.int32),
        pltpu.VMEM((CHUNK,), jnp.int32),
        pltpu.VMEM((CHUNK,), jnp.int32),
        pltpu.VMEM((CHUNK,), jnp.int32),
        pltpu.VMEM((CHUNK, SIZE), jnp.float32),
        pltpu.VMEM((CHUNK, SIZE), jnp.float32),
        pltpu.SemaphoreType.DMA,
        pltpu.SemaphoreType.DMA,
    ],
)
def _sc_lookup(idx_hbm, table_hbm, out_hbm,
               t1_sh, tv, idx0, idx1, cidx0, cidx1,
               rows0, rows1, gsem, wsem):
    c = lax.axis_index("c")
    s = lax.axis_index("s")
    wid = s * NUM_CORES + c

    @pl.when(s == 0)
    def _build():
        pltpu.sync_copy(table_hbm, tv.at[pl.ds(0, 9)])
        pltpu.sync_copy(tv, t1_sh)

    plsc.subcore_barrier()

    base0 = wid * B_PER_W
    rows = (rows0, rows1)
    idxs = (idx0, idx1)
    cidxs = (cidx0, cidx1)

    def prep(g):
        idx_v = idxs[g % 2]
        cidx_v = cidxs[g % 2]
        base = base0 + g * CHUNK
        pltpu.sync_copy(idx_hbm.at[pl.ds(base, CHUNK)], idx_v)

        def clamp_body(k, carry):
            v = idx_v[pl.ds(16 * k, 16)]
            cidx_v[pl.ds(16 * k, 16)] = (
                jnp.minimum(jnp.maximum(v, -K_CLIP), K_CLIP) + K_CLIP)
            return carry

        lax.fori_loop(0, CHUNK // 16, clamp_body, 0)

    def gather_start(g):
        pltpu.async_copy(t1_sh.at[cidxs[g % 2]], rows[g % 2], gsem)

    def gather_wait(g):
        pltpu.make_async_copy(t1_sh.at[cidxs[g % 2]], rows[g % 2],
                              gsem).wait()

    def write_start(g):
        base = base0 + g * CHUNK
        pltpu.async_copy(rows[g % 2], out_hbm.at[pl.ds(base, CHUNK)], wsem)

    def write_wait(g):
        base = base0 + g * CHUNK
        pltpu.make_async_copy(rows[g % 2], out_hbm.at[pl.ds(base, CHUNK)],
                              wsem).wait()

    prep(0)
    gather_start(0)
    prep(1)
    gather_start(1)
    for g in range(N_CHUNKS):
        gather_wait(g)
        write_start(g)
        if g + 2 < N_CHUNKS:
            prep(g + 2)
            write_wait(g)
            gather_start(g + 2)
    write_wait(N_CHUNKS - 2)
    write_wait(N_CHUNKS - 1)


def kernel(input, table):
    out = _sc_lookup(input.reshape(-1), table)
    return out.reshape(BATCH, HIST, SIZE)
